# baseline (speedup 1.0000x reference)
import functools

import jax
import jax.numpy as jnp
from jax import lax
from jax.experimental import pallas as pl
from jax.experimental.pallas import tpu as pltpu
from jax.experimental.pallas import tpu_sc as plsc

BATCH = 16384
D = 64
H1 = 128
CHUNK = 128


def _sc_gather(uid2, iid2, U1, Q1, A1f, B1f):
    mesh = plsc.VectorSubcoreMesh(core_axis_name="c", subcore_axis_name="s")
    nw = mesh.num_cores * mesh.num_subcores
    bpw = BATCH // nw
    nch = bpw // CHUNK

    @functools.partial(
        pl.kernel,
        out_type=[
            jax.ShapeDtypeStruct((BATCH, D), jnp.float32),
            jax.ShapeDtypeStruct((BATCH, D), jnp.float32),
            jax.ShapeDtypeStruct((BATCH,), jnp.float32),
            jax.ShapeDtypeStruct((BATCH,), jnp.float32),
        ],
        mesh=mesh,
        scratch_types=[
            pltpu.VMEM((nch, CHUNK), jnp.int32),
            pltpu.VMEM((nch, CHUNK), jnp.int32),
            pltpu.VMEM((bpw, D), jnp.float32),
            pltpu.VMEM((bpw, D), jnp.float32),
            pltpu.VMEM((bpw,), jnp.float32),
            pltpu.VMEM((bpw,), jnp.float32),
            pltpu.SemaphoreType.DMA,
        ],
        compiler_params=pltpu.CompilerParams(use_tc_tiling_on_sc=False),
    )
    def k(uid_h, iid_h, u_h, q_h, a_h, b_h, ue_h, ie_h, ub_h, ib_h,
          uidx, iidx, urows, qrows, ubv, ibv, sem):
        wid = lax.axis_index("s") * mesh.num_cores + lax.axis_index("c")
        base = wid * bpw
        row0 = wid * nch
        pltpu.sync_copy(uid_h.at[pl.ds(row0, nch)], uidx)
        pltpu.sync_copy(iid_h.at[pl.ds(row0, nch)], iidx)
        copies = []
        for j in range(nch):
            o = j * CHUNK
            copies.append(pltpu.async_copy(
                u_h.at[uidx.at[j]], urows.at[pl.ds(o, CHUNK)], sem))
            copies.append(pltpu.async_copy(
                q_h.at[iidx.at[j]], qrows.at[pl.ds(o, CHUNK)], sem))
            copies.append(pltpu.async_copy(
                a_h.at[uidx.at[j]], ubv.at[pl.ds(o, CHUNK)], sem))
            copies.append(pltpu.async_copy(
                b_h.at[iidx.at[j]], ibv.at[pl.ds(o, CHUNK)], sem))
        for c in copies:
            c.wait()
        pltpu.sync_copy(urows, ue_h.at[pl.ds(base, bpw)])
        pltpu.sync_copy(qrows, ie_h.at[pl.ds(base, bpw)])
        pltpu.sync_copy(ubv, ub_h.at[pl.ds(base, bpw)])
        pltpu.sync_copy(ibv, ib_h.at[pl.ds(base, bpw)])

    return k(uid2, iid2, U1, Q1, A1f, B1f)


def _tc_body(u_ref, q_ref, ub_ref, ib_ref, w1_ref, b1_ref, w2_ref, b2_ref,
             pred_ref, score_ref):
    u = u_ref[...]
    q = q_ref[...]
    p = u * q
    pred_ref[...] = jnp.sum(p, axis=1) + ub_ref[...] + ib_ref[...]
    x = jnp.concatenate([u, q, p], axis=1)
    h = lax.dot_general(x, w1_ref[...], (((1,), (1,)), ((), ())),
                        preferred_element_type=jnp.float32)
    h = jnp.maximum(h + b1_ref[...][None, :], 0.0)
    score_ref[...] = jnp.sum(h * w2_ref[...][None, :], axis=1) + b2_ref[0]


def _tc_dense(ue, ie, ub, ib, W1, b1, W2r, b2):
    bb = 2048
    grid = (BATCH // bb,)
    return pl.pallas_call(
        _tc_body,
        grid=grid,
        in_specs=[
            pl.BlockSpec((bb, D), lambda i: (i, 0)),
            pl.BlockSpec((bb, D), lambda i: (i, 0)),
            pl.BlockSpec((bb,), lambda i: (i,)),
            pl.BlockSpec((bb,), lambda i: (i,)),
            pl.BlockSpec((H1, 3 * D), lambda i: (0, 0)),
            pl.BlockSpec((H1,), lambda i: (0,)),
            pl.BlockSpec((H1,), lambda i: (0,)),
            pl.BlockSpec(memory_space=pltpu.SMEM),
        ],
        out_specs=[
            pl.BlockSpec((bb,), lambda i: (i,)),
            pl.BlockSpec((bb,), lambda i: (i,)),
        ],
        out_shape=[
            jax.ShapeDtypeStruct((BATCH,), jnp.float32),
            jax.ShapeDtypeStruct((BATCH,), jnp.float32),
        ],
    )(ue, ie, ub, ib, W1, b1, W2r, b2)


def kernel(user_ids, item_ids, U1, Q1, A1, B1, W1, b1, W2, b2):
    uid2 = user_ids.astype(jnp.int32).reshape(BATCH // CHUNK, CHUNK)
    iid2 = item_ids.astype(jnp.int32).reshape(BATCH // CHUNK, CHUNK)
    ue, ie, ub, ib = _sc_gather(uid2, iid2, U1, Q1,
                                A1.reshape(-1), B1.reshape(-1))
    pred, score = _tc_dense(ue, ie, ub, ib, W1, b1, W2.reshape(-1), b2)
    return (pred, score)


# speedup vs baseline: 1.7704x; 1.7704x over previous
import functools

import jax
import jax.numpy as jnp
from jax import lax
from jax.experimental import pallas as pl
from jax.experimental.pallas import tpu as pltpu
from jax.experimental.pallas import tpu_sc as plsc

BATCH = 16384
D = 64
H1 = 128
V = 100000
FPS = 4
OCHUNK = 8192


def _lookup_all(ids_v, slice_v, out_v, out_h, f, col0):
    def body(i, _):
        iv = ids_v[pl.ds(col0 + i * 16, 16)]
        out_v[pl.ds(i * 16, 16)] = plsc.load_gather(slice_v, [iv])
        return 0

    lax.fori_loop(0, OCHUNK // 16, body, 0, unroll=8)
    pltpu.sync_copy(out_v, out_h.at[f, pl.ds(col0, OCHUNK)])


def _do_table(tbl_h, bias_h, ids_h, emb_h, bias_out_h, s,
              ids_v, slice_v, out_v):
    pltpu.sync_copy(ids_h, ids_v)
    for k in range(FPS):
        f = s * FPS + k
        pltpu.sync_copy(tbl_h.at[f], slice_v)
        for half in range(BATCH // OCHUNK):
            _lookup_all(ids_v, slice_v, out_v, emb_h, f, half * OCHUNK)

    @pl.when(s == 15)
    def _bias():
        pltpu.sync_copy(bias_h.at[0], slice_v)

        for half in range(BATCH // OCHUNK):
            col0 = half * OCHUNK

            def body(i, _):
                iv = ids_v[pl.ds(col0 + i * 16, 16)]
                out_v[pl.ds(i * 16, 16)] = plsc.load_gather(slice_v, [iv])
                return 0

            lax.fori_loop(0, OCHUNK // 16, body, 0, unroll=8)
            pltpu.sync_copy(out_v, bias_out_h.at[pl.ds(col0, OCHUNK)])


def _sc_gather(uid, iid, U1t, Q1t, A1t, B1t):
    mesh = plsc.VectorSubcoreMesh(core_axis_name="c", subcore_axis_name="s")

    @functools.partial(
        pl.kernel,
        out_type=[
            jax.ShapeDtypeStruct((D, BATCH), jnp.float32),
            jax.ShapeDtypeStruct((D, BATCH), jnp.float32),
            jax.ShapeDtypeStruct((BATCH,), jnp.float32),
            jax.ShapeDtypeStruct((BATCH,), jnp.float32),
        ],
        mesh=mesh,
        scratch_types=[
            pltpu.VMEM((BATCH,), jnp.int32),
            pltpu.VMEM((V,), jnp.float32),
            pltpu.VMEM((OCHUNK,), jnp.float32),
        ],
        compiler_params=pltpu.CompilerParams(needs_layout_passes=False),
    )
    def k(uid_h, iid_h, u_h, q_h, a_h, b_h, ueT_h, ieT_h, ub_h, ib_h,
          ids_v, slice_v, out_v):
        s = lax.axis_index("s")
        c = lax.axis_index("c")

        @pl.when(c == 0)
        def _user():
            _do_table(u_h, a_h, uid_h, ueT_h, ub_h, s, ids_v, slice_v, out_v)

        @pl.when(c == 1)
        def _item():
            _do_table(q_h, b_h, iid_h, ieT_h, ib_h, s, ids_v, slice_v, out_v)

    return k(uid, iid, U1t, Q1t, A1t, B1t)


def _tc_body(uT_ref, qT_ref, ub_ref, ib_ref, w1_ref, b1_ref, w2_ref, b2_ref,
             pred_ref, score_ref):
    uT = uT_ref[...]
    qT = qT_ref[...]
    pT = uT * qT
    pred_ref[...] = jnp.sum(pT, axis=0) + ub_ref[...] + ib_ref[...]
    xT = jnp.concatenate([uT, qT, pT], axis=0)
    h = lax.dot_general(w1_ref[...], xT, (((1,), (0,)), ((), ())),
                        preferred_element_type=jnp.float32)
    h = jnp.maximum(h + b1_ref[...][:, None], 0.0)
    score_ref[...] = jnp.sum(h * w2_ref[...][:, None], axis=0) + b2_ref[0]


def _tc_dense(ueT, ieT, ub, ib, W1, b1, W2r, b2):
    bb = 2048
    grid = (BATCH // bb,)
    return pl.pallas_call(
        _tc_body,
        grid=grid,
        in_specs=[
            pl.BlockSpec((D, bb), lambda i: (0, i)),
            pl.BlockSpec((D, bb), lambda i: (0, i)),
            pl.BlockSpec((bb,), lambda i: (i,)),
            pl.BlockSpec((bb,), lambda i: (i,)),
            pl.BlockSpec((H1, 3 * D), lambda i: (0, 0)),
            pl.BlockSpec((H1,), lambda i: (0,)),
            pl.BlockSpec((H1,), lambda i: (0,)),
            pl.BlockSpec(memory_space=pltpu.SMEM),
        ],
        out_specs=[
            pl.BlockSpec((bb,), lambda i: (i,)),
            pl.BlockSpec((bb,), lambda i: (i,)),
        ],
        out_shape=[
            jax.ShapeDtypeStruct((BATCH,), jnp.float32),
            jax.ShapeDtypeStruct((BATCH,), jnp.float32),
        ],
    )(ueT, ieT, ub, ib, W1, b1, W2r, b2)


def kernel(user_ids, item_ids, U1, Q1, A1, B1, W1, b1, W2, b2):
    uid = user_ids.astype(jnp.int32)
    iid = item_ids.astype(jnp.int32)
    ueT, ieT, ub, ib = _sc_gather(uid, iid, U1.T, Q1.T, A1.T, B1.T)
    pred, score = _tc_dense(ueT, ieT, ub, ib, W1, b1, W2.reshape(-1), b2)
    return (pred, score)


# speedup vs baseline: 2.7642x; 1.5614x over previous
import functools

import jax
import jax.numpy as jnp
from jax import lax
from jax.experimental import pallas as pl
from jax.experimental.pallas import tpu as pltpu
from jax.experimental.pallas import tpu_sc as plsc

BATCH = 16384
D = 64
H1 = 128
V = 100000
FPS = 4
OCHUNK = 8192


def _lookup_all(ids_v, slice_v, out_v, out_h, f, col0):
    @plsc.parallel_loop(0, OCHUNK // 16, step=1, unroll=8)
    def body(i):
        iv = ids_v[pl.ds(col0 + i * 16, 16)]
        out_v[pl.ds(i * 16, 16)] = plsc.load_gather(slice_v, [iv])

    pltpu.sync_copy(out_v, out_h.at[f, pl.ds(col0, OCHUNK)])


def _do_table(tbl_h, bias_h, ids_h, emb_h, bias_out_h, s,
              ids_v, slice_v, out_v):
    pltpu.sync_copy(ids_h, ids_v)
    for k in range(FPS):
        f = s * FPS + k
        pltpu.sync_copy(tbl_h.at[f], slice_v)
        for half in range(BATCH // OCHUNK):
            _lookup_all(ids_v, slice_v, out_v, emb_h, f, half * OCHUNK)

    @pl.when(s == 15)
    def _bias():
        pltpu.sync_copy(bias_h.at[0], slice_v)

        for half in range(BATCH // OCHUNK):
            col0 = half * OCHUNK

            @plsc.parallel_loop(0, OCHUNK // 16, step=1, unroll=8)
            def body(i):
                iv = ids_v[pl.ds(col0 + i * 16, 16)]
                out_v[pl.ds(i * 16, 16)] = plsc.load_gather(slice_v, [iv])

            pltpu.sync_copy(out_v, bias_out_h.at[pl.ds(col0, OCHUNK)])


def _sc_gather(uid, iid, U1t, Q1t, A1t, B1t):
    mesh = plsc.VectorSubcoreMesh(core_axis_name="c", subcore_axis_name="s")

    @functools.partial(
        pl.kernel,
        out_type=[
            jax.ShapeDtypeStruct((D, BATCH), jnp.float32),
            jax.ShapeDtypeStruct((D, BATCH), jnp.float32),
            jax.ShapeDtypeStruct((BATCH,), jnp.float32),
            jax.ShapeDtypeStruct((BATCH,), jnp.float32),
        ],
        mesh=mesh,
        scratch_types=[
            pltpu.VMEM((BATCH,), jnp.int32),
            pltpu.VMEM((V,), jnp.float32),
            pltpu.VMEM((OCHUNK,), jnp.float32),
        ],
        compiler_params=pltpu.CompilerParams(needs_layout_passes=False),
    )
    def k(uid_h, iid_h, u_h, q_h, a_h, b_h, ueT_h, ieT_h, ub_h, ib_h,
          ids_v, slice_v, out_v):
        s = lax.axis_index("s")
        c = lax.axis_index("c")

        @pl.when(c == 0)
        def _user():
            _do_table(u_h, a_h, uid_h, ueT_h, ub_h, s, ids_v, slice_v, out_v)

        @pl.when(c == 1)
        def _item():
            _do_table(q_h, b_h, iid_h, ieT_h, ib_h, s, ids_v, slice_v, out_v)

    return k(uid, iid, U1t, Q1t, A1t, B1t)


def _tc_body(uT_ref, qT_ref, ub_ref, ib_ref, w1_ref, b1_ref, w2_ref, b2_ref,
             pred_ref, score_ref):
    uT = uT_ref[...]
    qT = qT_ref[...]
    pT = uT * qT
    pred_ref[...] = jnp.sum(pT, axis=0) + ub_ref[...] + ib_ref[...]
    xT = jnp.concatenate([uT, qT, pT], axis=0)
    h = lax.dot_general(w1_ref[...], xT, (((1,), (0,)), ((), ())),
                        preferred_element_type=jnp.float32)
    h = jnp.maximum(h + b1_ref[...][:, None], 0.0)
    score_ref[...] = jnp.sum(h * w2_ref[...][:, None], axis=0) + b2_ref[0]


def _tc_dense(ueT, ieT, ub, ib, W1, b1, W2r, b2):
    bb = 2048
    grid = (BATCH // bb,)
    return pl.pallas_call(
        _tc_body,
        grid=grid,
        in_specs=[
            pl.BlockSpec((D, bb), lambda i: (0, i)),
            pl.BlockSpec((D, bb), lambda i: (0, i)),
            pl.BlockSpec((bb,), lambda i: (i,)),
            pl.BlockSpec((bb,), lambda i: (i,)),
            pl.BlockSpec((H1, 3 * D), lambda i: (0, 0)),
            pl.BlockSpec((H1,), lambda i: (0,)),
            pl.BlockSpec((H1,), lambda i: (0,)),
            pl.BlockSpec(memory_space=pltpu.SMEM),
        ],
        out_specs=[
            pl.BlockSpec((bb,), lambda i: (i,)),
            pl.BlockSpec((bb,), lambda i: (i,)),
        ],
        out_shape=[
            jax.ShapeDtypeStruct((BATCH,), jnp.float32),
            jax.ShapeDtypeStruct((BATCH,), jnp.float32),
        ],
    )(ueT, ieT, ub, ib, W1, b1, W2r, b2)


def kernel(user_ids, item_ids, U1, Q1, A1, B1, W1, b1, W2, b2):
    uid = user_ids.astype(jnp.int32)
    iid = item_ids.astype(jnp.int32)
    ueT, ieT, ub, ib = _sc_gather(uid, iid, U1.T, Q1.T, A1.T, B1.T)
    pred, score = _tc_dense(ueT, ieT, ub, ib, W1, b1, W2.reshape(-1), b2)
    return (pred, score)
